# Initial kernel scaffold; baseline (speedup 1.0000x reference)
import jax
import jax.numpy as jnp
from jax.experimental import pallas as pl


def kernel(z, codebook):
    raise NotImplementedError("write your pallas kernel here")


# speedup vs baseline: 22.7174x; 22.7174x over previous
import functools

import jax
import jax.numpy as jnp
from jax.experimental import pallas as pl
from jax.experimental.pallas import tpu as pltpu

_B, _T, _D = 8, 1024, 256
_P = 4
_K = 1024
_DP = _D // _P
_BETA = 0.25
_N = _B * _T
_NBLK = 1024


def _vq_loss_kernel(z_ref, ct_ref, out_ref):
    i = pl.program_id(0)
    zb = z_ref[...]
    acc = jnp.sum(zb * zb)
    zbf = zb.astype(jnp.bfloat16)
    for p in range(_P):
        ct = ct_ref[p]
        cnorm = jnp.sum(ct * ct, axis=0)
        g = jax.lax.dot_general(
            zbf[:, p * _DP:(p + 1) * _DP],
            ct.astype(jnp.bfloat16),
            (((1,), (0,)), ((), ())),
            preferred_element_type=jnp.float32,
        )
        m = jnp.min(cnorm[None, :] - 2.0 * g, axis=1)
        acc = acc + jnp.sum(m)

    part = (acc * ((1.0 + _BETA) / (_B * _T * _D)))[None, None]

    @pl.when(i == 0)
    def _():
        out_ref[...] = jnp.zeros((1, 1), jnp.float32)

    out_ref[...] += part


@functools.partial(jax.jit, static_argnames=())
def kernel(z, codebook):
    z2 = z.reshape(_N, _D)
    ct = codebook.transpose(0, 2, 1)
    out = pl.pallas_call(
        _vq_loss_kernel,
        grid=(_N // _NBLK,),
        in_specs=[
            pl.BlockSpec((_NBLK, _D), lambda i: (i, 0)),
            pl.BlockSpec((_P, _DP, _K), lambda i: (0, 0, 0)),
        ],
        out_specs=pl.BlockSpec((1, 1), lambda i: (0, 0)),
        out_shape=jax.ShapeDtypeStruct((1, 1), jnp.float32),
    )(z2, ct)
    return out[0, 0]
